# baseline (speedup 1.0000x reference)
import functools

import jax
import jax.numpy as jnp
from jax import lax
from jax.experimental import pallas as pl
from jax.experimental.pallas import tpu as pltpu
from jax.experimental.pallas import tpu_sc as plsc

_N = 10000
_C = 128
_E = 320000
_EPS = 1e-5

_NC = 2
_NS = 16
_H = _C // 2
_K = 512
_EPT = _E // _NS
_FULL = _EPT // _K
_REM = _EPT - _FULL * _K
_EPAD = 2 * _E + 2 * _K
_NP = _N
_RPT = 624
_TAIL = _N - _RPT * _NS

_mesh = plsc.VectorSubcoreMesh(core_axis_name="c", subcore_axis_name="s")


@functools.partial(
    pl.kernel,
    out_type=jax.ShapeDtypeStruct((_NC, _N, _H), jnp.float32),
    mesh=_mesh,
    compiler_params=pltpu.CompilerParams(use_tc_tiling_on_sc=False),
    scratch_types=[
        pltpu.VMEM((_K,), jnp.int32),
        pltpu.VMEM((_K,), jnp.int32),
        pltpu.VMEM((_K,), jnp.int32),
        pltpu.VMEM((_K,), jnp.int32),
        pltpu.VMEM((_K, _H), jnp.float32),
        pltpu.VMEM((_K, _H), jnp.float32),
        pltpu.VMEM((_REM,), jnp.int32),
        pltpu.VMEM((_REM,), jnp.int32),
        pltpu.VMEM((_REM, _H), jnp.float32),
        pltpu.VMEM_SHARED((_NP, _H), jnp.float32),
        pltpu.SemaphoreType.DMA,
        pltpu.SemaphoreType.DMA,
        pltpu.SemaphoreType.DMA,
        pltpu.SemaphoreType.DMA,
    ],
)
def _sc_segment_sum(srce_ref, dste_ref, wedge_ref, zeros_ref, out_ref,
                    si0, si1, di0, di1, rows0, rows1, sit, dit, rowst,
                    acc_sh, gi0, gi1, g0, g1):
    cid = lax.axis_index("c")
    sid = lax.axis_index("s")
    e0 = cid * _E + sid * _EPT

    def idx_start(c, si, di, sem):
        base = e0 + c * _K
        pltpu.async_copy(srce_ref.at[pl.ds(base, _K)], si, sem)
        pltpu.async_copy(dste_ref.at[pl.ds(base, _K)], di, sem)

    def idx_wait(si, di, sem):
        pltpu.make_async_copy(srce_ref.at[pl.ds(0, _K)], si, sem).wait()
        pltpu.make_async_copy(dste_ref.at[pl.ds(0, _K)], di, sem).wait()

    idx_start(0, si0, di0, gi0)
    idx_start(1, si1, di1, gi1)

    r0 = sid * _RPT
    pltpu.sync_copy(zeros_ref.at[pl.ds(r0, _RPT)], acc_sh.at[pl.ds(r0, _RPT)])

    @pl.when(sid == _NS - 1)
    def _():
        t0 = _RPT * _NS
        pltpu.sync_copy(zeros_ref.at[pl.ds(t0, _NP - t0)],
                        acc_sh.at[pl.ds(t0, _NP - t0)])

    idx_wait(si0, di0, gi0)
    pltpu.async_copy(wedge_ref.at[si0], rows0, g0)
    plsc.subcore_barrier()

    def body(i, carry):
        c = 2 * i
        pltpu.make_async_copy(wedge_ref.at[si0], rows0, g0).wait()
        idx_wait(si1, di1, gi1)
        pltpu.async_copy(wedge_ref.at[si1], rows1, g1)
        pltpu.sync_copy(rows0, acc_sh.at[di0], add=True)
        idx_start(c + 2, si0, di0, gi0)
        pltpu.make_async_copy(wedge_ref.at[si1], rows1, g1).wait()
        idx_wait(si0, di0, gi0)
        pltpu.async_copy(wedge_ref.at[si0], rows0, g0)
        pltpu.sync_copy(rows1, acc_sh.at[di1], add=True)
        idx_start(c + 3, si1, di1, gi1)
        return carry

    lax.fori_loop(0, _FULL // 2, body, 0)
    pltpu.make_async_copy(wedge_ref.at[si0], rows0, g0).wait()
    pltpu.sync_copy(rows0, acc_sh.at[di0], add=True)
    idx_wait(si1, di1, gi1)

    tbase = e0 + _FULL * _K
    pltpu.sync_copy(srce_ref.at[pl.ds(tbase, _REM)], sit)
    pltpu.sync_copy(dste_ref.at[pl.ds(tbase, _REM)], dit)
    pltpu.async_copy(wedge_ref.at[sit], rowst, g0).wait()
    pltpu.sync_copy(rowst, acc_sh.at[dit], add=True)

    plsc.subcore_barrier()
    pltpu.sync_copy(acc_sh.at[pl.ds(r0, _RPT)],
                    out_ref.at[cid, pl.ds(r0, _RPT)])

    @pl.when(sid == _NS - 1)
    def _():
        t0 = _RPT * _NS
        pltpu.sync_copy(acc_sh.at[pl.ds(t0, _TAIL)],
                        out_ref.at[cid, pl.ds(t0, _TAIL)])


def _mm(a, b):
    return lax.dot_general(a, b, (((1,), (1,)), ((), ())),
                           preferred_element_type=jnp.float32)


def _style(h_in, sty, lW, lb, aWg, aWb, abg, abb, nzs):
    h = _mm(h_in, lW) + lb + nzs
    gamma = _mm(sty, aWg) + abg
    beta = _mm(sty, aWb) + abb
    mu = jnp.mean(h, axis=0, keepdims=True)
    var = jnp.mean((h - mu) * (h - mu), axis=0, keepdims=True)
    hn = (h - mu) * lax.rsqrt(var + _EPS)
    h = gamma * hn + beta
    return jnp.where(h >= 0, h, 0.01 * h)


def _tc1_body(x_ref, style_ref, l0W_ref, l0b_ref, a0Wg_ref, a0Wb_ref,
              a0bg_ref, a0bb_ref, nzs0_ref, xm_ref):
    xm_ref[...] = _style(x_ref[...], style_ref[...], l0W_ref[...],
                         l0b_ref[...], a0Wg_ref[...], a0Wb_ref[...],
                         a0bg_ref[...], a0bb_ref[...], nzs0_ref[...])


_tc1_call = pl.pallas_call(
    _tc1_body,
    out_shape=jax.ShapeDtypeStruct((_N, _C), jnp.float32),
)


def _tc2_body(acc_ref, xm_ref, style_ref, bedge_ref, Wc1_ref, bc1_ref,
              Wc2_ref, bc2_ref,
              l1W_ref, l1b_ref, a1Wg_ref, a1Wb_ref, a1bg_ref, a1bb_ref,
              nzs1_ref,
              l2W_ref, l2b_ref, a2Wg_ref, a2Wb_ref, a2bg_ref, a2bb_ref,
              nzs2_ref, out_ref):
    sty = style_ref[...]
    xm = xm_ref[...]
    out = jnp.concatenate([acc_ref[0], acc_ref[1]], axis=1) + bedge_ref[...]
    out = out + _mm(out, Wc1_ref[...]) + bc1_ref[...]
    out = out + xm
    out = out + _mm(xm, Wc2_ref[...]) + bc2_ref[...]
    out = jnp.maximum(out, 0.0)
    out = _style(out, sty, l1W_ref[...], l1b_ref[...], a1Wg_ref[...],
                 a1Wb_ref[...], a1bg_ref[...], a1bb_ref[...], nzs1_ref[...])
    out = _style(out, sty, l2W_ref[...], l2b_ref[...], a2Wg_ref[...],
                 a2Wb_ref[...], a2bg_ref[...], a2bb_ref[...], nzs2_ref[...])
    out_ref[...] = out


_tc2_call = pl.pallas_call(
    _tc2_body,
    out_shape=jax.ShapeDtypeStruct((_N, _C), jnp.float32),
)


def kernel(x, edge_index, style, W_edge, b_edge, Wc1, bc1, Wc2, bc2,
           l0W, l0b, a0W, a0b, ns0, nz0,
           l1W, l1b, a1W, a1b, ns1, nz1,
           l2W, l2b, a2W, a2b, ns2, nz2):
    ei = edge_index.astype(jnp.int32)
    npad = _EPAD - 2 * _E
    srcp = jnp.concatenate([ei[0], ei[0] + _N, jnp.zeros((npad,), jnp.int32)])
    dstp = jnp.concatenate([ei[1], ei[1], jnp.zeros((npad,), jnp.int32)])
    wedge_lr = jnp.concatenate([W_edge[:, :_H], W_edge[:, _H:]], axis=0)
    zeros = jnp.zeros((_NP, _H), jnp.float32)
    acc = _sc_segment_sum(srcp, dstp, wedge_lr, zeros)

    def prep(aW, ab, ns, nz):
        return (aW[:_C], aW[_C:], ab[:_C].reshape(1, _C),
                ab[_C:].reshape(1, _C), (ns * nz).reshape(1, _C))

    a0Wg, a0Wb, a0bg, a0bb, nzs0 = prep(a0W, a0b, ns0, nz0)
    a1Wg, a1Wb, a1bg, a1bb, nzs1 = prep(a1W, a1b, ns1, nz1)
    a2Wg, a2Wb, a2bg, a2bb, nzs2 = prep(a2W, a2b, ns2, nz2)

    xm = _tc1_call(x, style, l0W, l0b.reshape(1, _C), a0Wg, a0Wb,
                   a0bg, a0bb, nzs0)

    return _tc2_call(
        acc, xm, style, b_edge.reshape(1, _C), Wc1, bc1.reshape(1, _C),
        Wc2, bc2.reshape(1, _C),
        l1W, l1b.reshape(1, _C), a1Wg, a1Wb, a1bg, a1bb, nzs1,
        l2W, l2b.reshape(1, _C), a2Wg, a2Wb, a2bg, a2bb, nzs2)


# speedup vs baseline: 1.0544x; 1.0544x over previous
import functools

import jax
import jax.numpy as jnp
from jax import lax
from jax.experimental import pallas as pl
from jax.experimental.pallas import tpu as pltpu
from jax.experimental.pallas import tpu_sc as plsc

_N = 10000
_C = 128
_E = 320000
_EPS = 1e-5

_NC = 2
_NS = 16
_NW = _NC * _NS
_K = 128
_EPW = _E // _NW
_FULL = 78
_REM = _EPW - _FULL * _K
_EPAD = _E + 2 * _K
_NP = _N
_RPT = 624
_TAIL = _N - _RPT * _NS

_mesh = plsc.VectorSubcoreMesh(core_axis_name="c", subcore_axis_name="s")


@functools.partial(
    pl.kernel,
    out_type=jax.ShapeDtypeStruct((_NC, _N, _C), jnp.float32),
    mesh=_mesh,
    scratch_types=[
        pltpu.VMEM((_K,), jnp.int32),
        pltpu.VMEM((_K,), jnp.int32),
        pltpu.VMEM((_K,), jnp.int32),
        pltpu.VMEM((_K,), jnp.int32),
        pltpu.VMEM((_K, _C), jnp.float32),
        pltpu.VMEM((_K, _C), jnp.float32),
        pltpu.VMEM((_REM,), jnp.int32),
        pltpu.VMEM((_REM,), jnp.int32),
        pltpu.VMEM((_REM, _C), jnp.float32),
        pltpu.VMEM_SHARED((_NP, _C), jnp.float32),
        pltpu.SemaphoreType.DMA,
        pltpu.SemaphoreType.DMA,
        pltpu.SemaphoreType.DMA,
        pltpu.SemaphoreType.DMA,
    ],
)
def _sc_segment_sum(srce_ref, dste_ref, wedge_ref, out_ref,
                    si0, si1, di0, di1, rows0, rows1, sit, dit, rowst,
                    acc_sh, gi0, gi1, g0, g1):
    cid = lax.axis_index("c")
    sid = lax.axis_index("s")
    wid = cid * _NS + sid
    e0 = wid * _EPW

    def idx_start(c, si, di, sem):
        base = e0 + c * _K
        pltpu.async_copy(srce_ref.at[pl.ds(base, _K)], si, sem)
        pltpu.async_copy(dste_ref.at[pl.ds(base, _K)], di, sem)

    def idx_wait(si, di, sem):
        pltpu.make_async_copy(srce_ref.at[pl.ds(0, _K)], si, sem).wait()
        pltpu.make_async_copy(dste_ref.at[pl.ds(0, _K)], di, sem).wait()

    idx_start(0, si0, di0, gi0)
    idx_start(1, si1, di1, gi1)
    idx_wait(si0, di0, gi0)
    pltpu.async_copy(wedge_ref.at[si0], rows0, g0)

    zv = jnp.zeros((16,), jnp.float32)

    def zfill(r, carry):
        for l in range(_C // 16):
            rows1[r, pl.ds(l * 16, 16)] = zv
        return carry

    lax.fori_loop(0, _K, zfill, 0)
    r0 = sid * _RPT
    for t in range(4):
        pltpu.sync_copy(rows1, acc_sh.at[pl.ds(r0 + t * _K, _K)])
    pltpu.sync_copy(rows1.at[pl.ds(0, _RPT - 4 * _K)],
                    acc_sh.at[pl.ds(r0 + 4 * _K, _RPT - 4 * _K)])

    @pl.when(sid == _NS - 1)
    def _():
        t0 = _RPT * _NS
        pltpu.sync_copy(rows1.at[pl.ds(0, _NP - t0)],
                        acc_sh.at[pl.ds(t0, _NP - t0)])

    plsc.subcore_barrier()

    def body(i, carry):
        c = 2 * i
        pltpu.make_async_copy(wedge_ref.at[si0], rows0, g0).wait()
        idx_wait(si1, di1, gi1)
        pltpu.async_copy(wedge_ref.at[si1], rows1, g1)
        pltpu.sync_copy(rows0, acc_sh.at[di0], add=True)
        idx_start(c + 2, si0, di0, gi0)
        pltpu.make_async_copy(wedge_ref.at[si1], rows1, g1).wait()
        idx_wait(si0, di0, gi0)
        pltpu.async_copy(wedge_ref.at[si0], rows0, g0)
        pltpu.sync_copy(rows1, acc_sh.at[di1], add=True)
        idx_start(c + 3, si1, di1, gi1)
        return carry

    lax.fori_loop(0, _FULL // 2, body, 0)
    pltpu.make_async_copy(wedge_ref.at[si0], rows0, g0).wait()
    idx_wait(si1, di1, gi1)

    tbase = e0 + _FULL * _K
    pltpu.sync_copy(srce_ref.at[pl.ds(tbase, _REM)], sit)
    pltpu.sync_copy(dste_ref.at[pl.ds(tbase, _REM)], dit)
    pltpu.async_copy(wedge_ref.at[sit], rowst, g0).wait()
    pltpu.sync_copy(rowst, acc_sh.at[dit], add=True)

    plsc.subcore_barrier()
    pltpu.sync_copy(acc_sh.at[pl.ds(r0, _RPT)],
                    out_ref.at[cid, pl.ds(r0, _RPT)])

    @pl.when(sid == _NS - 1)
    def _():
        t0 = _RPT * _NS
        pltpu.sync_copy(acc_sh.at[pl.ds(t0, _TAIL)],
                        out_ref.at[cid, pl.ds(t0, _TAIL)])


def _mm(a, b):
    return lax.dot_general(a, b, (((1,), (1,)), ((), ())),
                           preferred_element_type=jnp.float32)


def _style(h_in, sty, lW, lb, aWg, aWb, abg, abb, nzs):
    h = _mm(h_in, lW) + lb + nzs
    gamma = _mm(sty, aWg) + abg
    beta = _mm(sty, aWb) + abb
    mu = jnp.mean(h, axis=0, keepdims=True)
    var = jnp.mean((h - mu) * (h - mu), axis=0, keepdims=True)
    hn = (h - mu) * lax.rsqrt(var + _EPS)
    h = gamma * hn + beta
    return jnp.where(h >= 0, h, 0.01 * h)


def _tc1_body(x_ref, style_ref, l0W_ref, l0b_ref, a0Wg_ref, a0Wb_ref,
              a0bg_ref, a0bb_ref, nzs0_ref, xm_ref):
    xm_ref[...] = _style(x_ref[...], style_ref[...], l0W_ref[...],
                         l0b_ref[...], a0Wg_ref[...], a0Wb_ref[...],
                         a0bg_ref[...], a0bb_ref[...], nzs0_ref[...])


_tc1_call = pl.pallas_call(
    _tc1_body,
    out_shape=jax.ShapeDtypeStruct((_N, _C), jnp.float32),
)


def _tc2_body(acc_ref, xm_ref, style_ref, bedge_ref, Wc1_ref, bc1_ref,
              Wc2_ref, bc2_ref,
              l1W_ref, l1b_ref, a1Wg_ref, a1Wb_ref, a1bg_ref, a1bb_ref,
              nzs1_ref,
              l2W_ref, l2b_ref, a2Wg_ref, a2Wb_ref, a2bg_ref, a2bb_ref,
              nzs2_ref, out_ref):
    sty = style_ref[...]
    xm = xm_ref[...]
    out = acc_ref[0] + acc_ref[1] + bedge_ref[...]
    out = out + _mm(out, Wc1_ref[...]) + bc1_ref[...]
    out = out + xm
    out = out + _mm(xm, Wc2_ref[...]) + bc2_ref[...]
    out = jnp.maximum(out, 0.0)
    out = _style(out, sty, l1W_ref[...], l1b_ref[...], a1Wg_ref[...],
                 a1Wb_ref[...], a1bg_ref[...], a1bb_ref[...], nzs1_ref[...])
    out = _style(out, sty, l2W_ref[...], l2b_ref[...], a2Wg_ref[...],
                 a2Wb_ref[...], a2bg_ref[...], a2bb_ref[...], nzs2_ref[...])
    out_ref[...] = out


_tc2_call = pl.pallas_call(
    _tc2_body,
    out_shape=jax.ShapeDtypeStruct((_N, _C), jnp.float32),
)


def kernel(x, edge_index, style, W_edge, b_edge, Wc1, bc1, Wc2, bc2,
           l0W, l0b, a0W, a0b, ns0, nz0,
           l1W, l1b, a1W, a1b, ns1, nz1,
           l2W, l2b, a2W, a2b, ns2, nz2):
    ei = edge_index.astype(jnp.int32)
    npad = _EPAD - _E
    srcp = jnp.concatenate([ei[0], jnp.zeros((npad,), jnp.int32)])
    dstp = jnp.concatenate([ei[1], jnp.zeros((npad,), jnp.int32)])
    acc = _sc_segment_sum(srcp, dstp, W_edge)

    def prep(aW, ab, ns, nz):
        return (aW[:_C], aW[_C:], ab[:_C].reshape(1, _C),
                ab[_C:].reshape(1, _C), (ns * nz).reshape(1, _C))

    a0Wg, a0Wb, a0bg, a0bb, nzs0 = prep(a0W, a0b, ns0, nz0)
    a1Wg, a1Wb, a1bg, a1bb, nzs1 = prep(a1W, a1b, ns1, nz1)
    a2Wg, a2Wb, a2bg, a2bb, nzs2 = prep(a2W, a2b, ns2, nz2)

    xm = _tc1_call(x, style, l0W, l0b.reshape(1, _C), a0Wg, a0Wb,
                   a0bg, a0bb, nzs0)

    return _tc2_call(
        acc, xm, style, b_edge.reshape(1, _C), Wc1, bc1.reshape(1, _C),
        Wc2, bc2.reshape(1, _C),
        l1W, l1b.reshape(1, _C), a1Wg, a1Wb, a1bg, a1bb, nzs1,
        l2W, l2b.reshape(1, _C), a2Wg, a2Wb, a2bg, a2bb, nzs2)


# speedup vs baseline: 1.1985x; 1.1367x over previous
import functools

import jax
import jax.numpy as jnp
from jax import lax
from jax.experimental import pallas as pl
from jax.experimental.pallas import tpu as pltpu
from jax.experimental.pallas import tpu_sc as plsc

_N = 10000
_C = 128
_E = 320000
_EPS = 1e-5

_NC = 2
_NS = 16
_NW = _NC * _NS
_K = 128
_EPW = _E // _NW
_FULL = 78
_REM = _EPW - _FULL * _K
_EPAD = _E + 4 * _K
_NB = 3
_NI = 6
_NP = _N
_RPT = 624
_TAIL = _N - _RPT * _NS

_mesh = plsc.VectorSubcoreMesh(core_axis_name="c", subcore_axis_name="s")


@functools.partial(
    pl.kernel,
    out_type=jax.ShapeDtypeStruct((_NC, _N, _C), jnp.float32),
    mesh=_mesh,
    scratch_types=(
        [pltpu.VMEM((_K,), jnp.int32)] * _NI +
        [pltpu.VMEM((_K,), jnp.int32)] * _NI +
        [pltpu.VMEM((_K, _C), jnp.float32)] * _NB +
        [pltpu.VMEM((_REM,), jnp.int32)] * 2 +
        [pltpu.VMEM_SHARED((_NP, _C), jnp.float32)] +
        [pltpu.SemaphoreType.DMA] * (2 * _NB + _NI)
    ),
)
def _sc_segment_sum(srce_ref, dste_ref, wedge_ref, out_ref, *refs):
    si = refs[0:_NI]
    di = refs[_NI:2 * _NI]
    rows = refs[2 * _NI:2 * _NI + _NB]
    sit, dit = refs[2 * _NI + _NB:2 * _NI + _NB + 2]
    acc_sh = refs[2 * _NI + _NB + 2]
    sems = refs[2 * _NI + _NB + 3:]
    g = sems[0:_NB]
    ss = sems[_NB:2 * _NB]
    gi = sems[2 * _NB:]

    cid = lax.axis_index("c")
    sid = lax.axis_index("s")
    wid = cid * _NS + sid
    e0 = wid * _EPW

    def idx_start(c, q):
        base = e0 + c * _K
        pltpu.async_copy(srce_ref.at[pl.ds(base, _K)], si[q], gi[q])
        pltpu.async_copy(dste_ref.at[pl.ds(base, _K)], di[q], gi[q])

    def idx_wait(q):
        pltpu.make_async_copy(srce_ref.at[pl.ds(0, _K)], si[q], gi[q]).wait()
        pltpu.make_async_copy(dste_ref.at[pl.ds(0, _K)], di[q], gi[q]).wait()

    def g_start(b, q):
        pltpu.async_copy(wedge_ref.at[si[q]], rows[b], g[b])

    def g_wait(b, q):
        pltpu.make_async_copy(wedge_ref.at[si[q]], rows[b], g[b]).wait()

    def s_start(b, q):
        pltpu.async_copy(rows[b], acc_sh.at[di[q]], ss[b], add=True)

    def s_wait(b, q):
        pltpu.make_async_copy(rows[b], acc_sh.at[di[q]], ss[b]).wait()

    for c in range(4):
        idx_start(c, c)
    idx_wait(0)
    g_start(0, 0)
    idx_wait(1)
    g_start(1, 1)

    zv = jnp.zeros((16,), jnp.float32)

    def zfill(r, carry):
        for l in range(_C // 16):
            rows[2][r, pl.ds(l * 16, 16)] = zv
        return carry

    lax.fori_loop(0, _K, zfill, 0)
    r0 = sid * _RPT
    for t in range(4):
        pltpu.sync_copy(rows[2], acc_sh.at[pl.ds(r0 + t * _K, _K)])
    pltpu.sync_copy(rows[2].at[pl.ds(0, _RPT - 4 * _K)],
                    acc_sh.at[pl.ds(r0 + 4 * _K, _RPT - 4 * _K)])

    @pl.when(sid == _NS - 1)
    def _():
        t0 = _RPT * _NS
        pltpu.sync_copy(rows[2].at[pl.ds(0, _NP - t0)],
                        acc_sh.at[pl.ds(t0, _NP - t0)])

    plsc.subcore_barrier()

    def step(i, k):
        c = 6 * i + k
        g_wait(k % _NB, k)
        s_start(k % _NB, k)
        if k == 0:
            @pl.when(i > 0)
            def _():
                s_wait((k - 1) % _NB, (k - 1) % _NI)
        else:
            s_wait((k - 1) % _NB, (k - 1) % _NI)
        idx_wait((k + 2) % _NI)
        g_start((k + 2) % _NB, (k + 2) % _NI)
        idx_start(c + 4, (k + 4) % _NI)

    def body(i, carry):
        for k in range(6):
            step(i, k)
        return carry

    lax.fori_loop(0, _FULL // 6, body, 0)
    s_wait((_FULL - 1) % _NB, (_FULL - 1) % _NI)
    g_wait(_FULL % _NB, _FULL % _NI)
    g_wait((_FULL + 1) % _NB, (_FULL + 1) % _NI)
    idx_wait((_FULL + 2) % _NI)
    idx_wait((_FULL + 3) % _NI)

    tbase = e0 + _FULL * _K
    pltpu.sync_copy(srce_ref.at[pl.ds(tbase, _REM)], sit)
    pltpu.sync_copy(dste_ref.at[pl.ds(tbase, _REM)], dit)
    pltpu.async_copy(wedge_ref.at[sit], rows[0].at[pl.ds(0, _REM)],
                     g[0]).wait()
    pltpu.sync_copy(rows[0].at[pl.ds(0, _REM)], acc_sh.at[dit], add=True)

    plsc.subcore_barrier()
    pltpu.sync_copy(acc_sh.at[pl.ds(r0, _RPT)],
                    out_ref.at[cid, pl.ds(r0, _RPT)])

    @pl.when(sid == _NS - 1)
    def _():
        t0 = _RPT * _NS
        pltpu.sync_copy(acc_sh.at[pl.ds(t0, _TAIL)],
                        out_ref.at[cid, pl.ds(t0, _TAIL)])


def _mm(a, b):
    return lax.dot_general(a, b, (((1,), (1,)), ((), ())),
                           preferred_element_type=jnp.float32)


def _style(h_in, sty, lW, lb, aWg, aWb, abg, abb, nzs):
    h = _mm(h_in, lW) + lb + nzs
    gamma = _mm(sty, aWg) + abg
    beta = _mm(sty, aWb) + abb
    mu = jnp.mean(h, axis=0, keepdims=True)
    var = jnp.mean((h - mu) * (h - mu), axis=0, keepdims=True)
    hn = (h - mu) * lax.rsqrt(var + _EPS)
    h = gamma * hn + beta
    return jnp.where(h >= 0, h, 0.01 * h)


def _tc1_body(x_ref, style_ref, l0W_ref, l0b_ref, a0Wg_ref, a0Wb_ref,
              a0bg_ref, a0bb_ref, nzs0_ref, xm_ref):
    xm_ref[...] = _style(x_ref[...], style_ref[...], l0W_ref[...],
                         l0b_ref[...], a0Wg_ref[...], a0Wb_ref[...],
                         a0bg_ref[...], a0bb_ref[...], nzs0_ref[...])


_tc1_call = pl.pallas_call(
    _tc1_body,
    out_shape=jax.ShapeDtypeStruct((_N, _C), jnp.float32),
)


def _tc2_body(acc_ref, xm_ref, style_ref, bedge_ref, Wc1_ref, bc1_ref,
              Wc2_ref, bc2_ref,
              l1W_ref, l1b_ref, a1Wg_ref, a1Wb_ref, a1bg_ref, a1bb_ref,
              nzs1_ref,
              l2W_ref, l2b_ref, a2Wg_ref, a2Wb_ref, a2bg_ref, a2bb_ref,
              nzs2_ref, out_ref):
    sty = style_ref[...]
    xm = xm_ref[...]
    out = acc_ref[0] + acc_ref[1] + bedge_ref[...]
    out = out + _mm(out, Wc1_ref[...]) + bc1_ref[...]
    out = out + xm
    out = out + _mm(xm, Wc2_ref[...]) + bc2_ref[...]
    out = jnp.maximum(out, 0.0)
    out = _style(out, sty, l1W_ref[...], l1b_ref[...], a1Wg_ref[...],
                 a1Wb_ref[...], a1bg_ref[...], a1bb_ref[...], nzs1_ref[...])
    out = _style(out, sty, l2W_ref[...], l2b_ref[...], a2Wg_ref[...],
                 a2Wb_ref[...], a2bg_ref[...], a2bb_ref[...], nzs2_ref[...])
    out_ref[...] = out


_tc2_call = pl.pallas_call(
    _tc2_body,
    out_shape=jax.ShapeDtypeStruct((_N, _C), jnp.float32),
)


def kernel(x, edge_index, style, W_edge, b_edge, Wc1, bc1, Wc2, bc2,
           l0W, l0b, a0W, a0b, ns0, nz0,
           l1W, l1b, a1W, a1b, ns1, nz1,
           l2W, l2b, a2W, a2b, ns2, nz2):
    ei = edge_index.astype(jnp.int32)
    npad = _EPAD - _E
    srcp = jnp.concatenate([ei[0], jnp.zeros((npad,), jnp.int32)])
    dstp = jnp.concatenate([ei[1], jnp.zeros((npad,), jnp.int32)])
    acc = _sc_segment_sum(srcp, dstp, W_edge)

    def prep(aW, ab, ns, nz):
        return (aW[:_C], aW[_C:], ab[:_C].reshape(1, _C),
                ab[_C:].reshape(1, _C), (ns * nz).reshape(1, _C))

    a0Wg, a0Wb, a0bg, a0bb, nzs0 = prep(a0W, a0b, ns0, nz0)
    a1Wg, a1Wb, a1bg, a1bb, nzs1 = prep(a1W, a1b, ns1, nz1)
    a2Wg, a2Wb, a2bg, a2bb, nzs2 = prep(a2W, a2b, ns2, nz2)

    xm = _tc1_call(x, style, l0W, l0b.reshape(1, _C), a0Wg, a0Wb,
                   a0bg, a0bb, nzs0)

    return _tc2_call(
        acc, xm, style, b_edge.reshape(1, _C), Wc1, bc1.reshape(1, _C),
        Wc2, bc2.reshape(1, _C),
        l1W, l1b.reshape(1, _C), a1Wg, a1Wb, a1bg, a1bb, nzs1,
        l2W, l2b.reshape(1, _C), a2Wg, a2Wb, a2bg, a2bb, nzs2)
